# baseline (speedup 1.0000x reference)
import jax
import jax.numpy as jnp
from jax.experimental import pallas as pl
from jax.experimental.pallas import tpu as pltpu


def _rpq_kernel(x_ref, m_ref, w_ref, cb_ref, out_ref,
                best_ref, lab_ref, cnt_ref, caug_ref):
    i = pl.program_id(0)
    nt = pl.num_programs(0)
    T = x_ref.shape[0]
    NC = cb_ref.shape[0]
    Lv = m_ref.shape[1]
    per_row = Lv // T

    @pl.when(i == 0)
    def _init():
        best_ref[0, 0] = jnp.inf
        lab_ref[0, 0] = 0
        cnt_ref[0, 0] = 0
        cb = cb_ref[...]
        ch = cb.astype(jnp.bfloat16).astype(jnp.float32)
        cl = cb - ch
        cn = jnp.sum(cb * cb, axis=1, keepdims=True)
        cnh = cn.astype(jnp.bfloat16).astype(jnp.float32)
        cnl = cn - cnh
        caug_ref[...] = jnp.concatenate(
            [ch * -2.0, cl * -2.0, ch * -2.0,
             jnp.ones((NC, 2), jnp.float32), cnh, cnl,
             jnp.zeros((NC, 4), jnp.float32)], axis=1)

    x = x_ref[...]
    t = jax.lax.dot_general(x, w_ref[...], (((1,), (1,)), ((), ())),
                            preferred_element_type=jnp.float32)
    th = t.astype(jnp.bfloat16).astype(jnp.float32)
    tl = t - th
    tn = jnp.sum(t * t, axis=1, keepdims=True)
    tnh = tn.astype(jnp.bfloat16).astype(jnp.float32)
    tnl = tn - tnh
    taug = jnp.concatenate(
        [th, th, tl, tnh, tnl, jnp.ones((T, 2), jnp.float32),
         jnp.zeros((T, 4), jnp.float32)], axis=1)
    d2t = jax.lax.dot_general(caug_ref[...], taug, (((1,), (1,)), ((), ())),
                              preferred_element_type=jnp.float32)

    moff = pl.multiple_of((i % per_row) * T, T)
    mlane = m_ref[pl.ds(i // per_row, 1), pl.ds(moff, T)] == 1
    colmin = jnp.min(d2t, axis=0, keepdims=True)
    vlane = jnp.where(mlane, colmin, jnp.inf)
    v = jnp.min(vlane)

    mi = mlane.astype(jnp.int32)
    cnt_here = cnt_ref[0, 0]
    cnt_ref[0, 0] = cnt_here + jnp.sum(mi)

    @pl.when(v < best_ref[0, 0])
    def _update():
        li = jax.lax.broadcasted_iota(jnp.int32, (1, T), 1)
        row = jnp.min(jnp.where(vlane == v, li, jnp.int32(2**31 - 1)))
        colv = jnp.min(jnp.where(li == row, d2t, jnp.inf), axis=1,
                       keepdims=True)
        si = jax.lax.broadcasted_iota(jnp.int32, (NC, 1), 0)
        col = jnp.min(jnp.where(colv == v, si, jnp.int32(2**31 - 1)))
        lrank = jnp.sum(jnp.where(li < row, mi, 0))
        best_ref[0, 0] = v
        lab_ref[0, 0] = (cnt_here + lrank) * NC + col

    @pl.when(i == nt - 1)
    def _fin():
        out_ref[0, 0] = lab_ref[0, 0]


def kernel(input_values, mask_time_indices, W, code_book):
    Bv, Lv, D = input_values.shape
    N = Bv * Lv
    K = W.shape[0]
    NC = code_book.shape[0]
    T = 1024
    while Lv % T:
        T //= 2
    x = input_values.reshape(N, D)
    out = pl.pallas_call(
        _rpq_kernel,
        grid=(N // T,),
        in_specs=[
            pl.BlockSpec((T, D), lambda i: (i, 0)),
            pl.BlockSpec((Bv, Lv), lambda i: (0, 0)),
            pl.BlockSpec((K, D), lambda i: (0, 0)),
            pl.BlockSpec((NC, K), lambda i: (0, 0)),
        ],
        out_specs=pl.BlockSpec((1, 1), lambda i: (0, 0),
                               memory_space=pltpu.SMEM),
        out_shape=jax.ShapeDtypeStruct((1, 1), jnp.int32),
        scratch_shapes=[
            pltpu.SMEM((1, 1), jnp.float32),
            pltpu.SMEM((1, 1), jnp.int32),
            pltpu.SMEM((1, 1), jnp.int32),
            pltpu.VMEM((NC, 3 * K + 8), jnp.float32),
        ],
    )(x, mask_time_indices, W, code_book)
    return out[0, 0]


# speedup vs baseline: 1.2117x; 1.2117x over previous
import jax
import jax.numpy as jnp
from jax.experimental import pallas as pl
from jax.experimental.pallas import tpu as pltpu


def _rpq_kernel(x_ref, m_ref, w_ref, cb_ref, out_ref,
                best_ref, lab_ref, cnt_ref, caug_ref):
    i = pl.program_id(0)
    nt = pl.num_programs(0)
    T = x_ref.shape[1]
    NC = cb_ref.shape[0]
    Lv = m_ref.shape[1]
    per_row = Lv // T

    @pl.when(i == 0)
    def _init():
        best_ref[0, 0] = jnp.inf
        lab_ref[0, 0] = 0
        cnt_ref[0, 0] = 0
        cb = cb_ref[...]
        ch = cb.astype(jnp.bfloat16).astype(jnp.float32)
        cl = cb - ch
        cn = jnp.sum(cb * cb, axis=1, keepdims=True)
        cnh = cn.astype(jnp.bfloat16).astype(jnp.float32)
        cnl = cn - cnh
        caug_ref[...] = jnp.concatenate(
            [ch * -2.0, cl * -2.0, ch * -2.0,
             jnp.ones((NC, 2), jnp.float32), cnh, cnl,
             jnp.zeros((NC, 4), jnp.float32)], axis=1)

    x = x_ref[...].reshape(T, x_ref.shape[2])
    t = jax.lax.dot_general(x, w_ref[...], (((1,), (1,)), ((), ())),
                            preferred_element_type=jnp.float32)
    th = t.astype(jnp.bfloat16).astype(jnp.float32)
    tl = t - th
    tn = jnp.sum(t * t, axis=1, keepdims=True)
    tnh = tn.astype(jnp.bfloat16).astype(jnp.float32)
    tnl = tn - tnh
    taug = jnp.concatenate(
        [th, th, tl, tnh, tnl, jnp.ones((T, 2), jnp.float32),
         jnp.zeros((T, 4), jnp.float32)], axis=1)
    d2t = jax.lax.dot_general(caug_ref[...], taug, (((1,), (1,)), ((), ())),
                              preferred_element_type=jnp.float32)

    moff = pl.multiple_of((i % per_row) * T, T)
    mlane = m_ref[pl.ds(i // per_row, 1), pl.ds(moff, T)] == 1
    colmin = jnp.min(d2t, axis=0, keepdims=True)
    vlane = jnp.where(mlane, colmin, jnp.inf)
    v = jnp.min(vlane)

    mi = mlane.astype(jnp.int32)
    cnt_here = cnt_ref[0, 0]
    cnt_ref[0, 0] = cnt_here + jnp.sum(mi)

    @pl.when(v < best_ref[0, 0])
    def _update():
        li = jax.lax.broadcasted_iota(jnp.int32, (1, T), 1)
        row = jnp.min(jnp.where(vlane == v, li, jnp.int32(2**31 - 1)))
        colv = jnp.min(jnp.where(li == row, d2t, jnp.inf), axis=1,
                       keepdims=True)
        si = jax.lax.broadcasted_iota(jnp.int32, (NC, 1), 0)
        col = jnp.min(jnp.where(colv == v, si, jnp.int32(2**31 - 1)))
        lrank = jnp.sum(jnp.where(li < row, mi, 0))
        best_ref[0, 0] = v
        lab_ref[0, 0] = (cnt_here + lrank) * NC + col

    @pl.when(i == nt - 1)
    def _fin():
        out_ref[0, 0] = lab_ref[0, 0]


def kernel(input_values, mask_time_indices, W, code_book):
    Bv, Lv, D = input_values.shape
    N = Bv * Lv
    K = W.shape[0]
    NC = code_book.shape[0]
    T = 1024
    while Lv % T:
        T //= 2
    pr = Lv // T
    out = pl.pallas_call(
        _rpq_kernel,
        grid=(N // T,),
        in_specs=[
            pl.BlockSpec((1, T, D), lambda i: (i // pr, i % pr, 0)),
            pl.BlockSpec((Bv, Lv), lambda i: (0, 0)),
            pl.BlockSpec((K, D), lambda i: (0, 0)),
            pl.BlockSpec((NC, K), lambda i: (0, 0)),
        ],
        out_specs=pl.BlockSpec((1, 1), lambda i: (0, 0),
                               memory_space=pltpu.SMEM),
        out_shape=jax.ShapeDtypeStruct((1, 1), jnp.int32),
        scratch_shapes=[
            pltpu.SMEM((1, 1), jnp.float32),
            pltpu.SMEM((1, 1), jnp.int32),
            pltpu.SMEM((1, 1), jnp.int32),
            pltpu.VMEM((NC, 3 * K + 8), jnp.float32),
        ],
    )(input_values, mask_time_indices, W, code_book)
    return out[0, 0]


# speedup vs baseline: 1.3883x; 1.1458x over previous
import jax
import jax.numpy as jnp
from jax.experimental import pallas as pl
from jax.experimental.pallas import tpu as pltpu


def _rpq_kernel(x_ref, m_ref, w_ref, cb_ref, out_ref,
                best_ref, lab_ref, cnt_ref, caug_ref):
    i = pl.program_id(0)
    nt = pl.num_programs(0)
    T = x_ref.shape[1]
    NC = cb_ref.shape[0]
    Lv = m_ref.shape[1]
    per_row = Lv // T

    @pl.when(i == 0)
    def _init():
        best_ref[0, 0] = jnp.inf
        lab_ref[0, 0] = 0
        cnt_ref[0, 0] = 0
        cb = cb_ref[...]
        ch = cb.astype(jnp.bfloat16).astype(jnp.float32)
        cl = cb - ch
        cn = jnp.sum(cb * cb, axis=1, keepdims=True)
        cnh = cn.astype(jnp.bfloat16).astype(jnp.float32)
        cnl = cn - cnh
        caug_ref[...] = jnp.concatenate(
            [ch * -2.0, cl * -2.0, ch * -2.0,
             jnp.ones((NC, 2), jnp.float32), cnh, cnl,
             jnp.zeros((NC, 4), jnp.float32)], axis=1)

    x = x_ref[...].reshape(T, x_ref.shape[2])
    t = jax.lax.dot_general(x, w_ref[...], (((1,), (1,)), ((), ())),
                            preferred_element_type=jnp.float32)
    th = t.astype(jnp.bfloat16).astype(jnp.float32)
    tl = t - th
    tn = jnp.sum(t * t, axis=1, keepdims=True)
    tnh = tn.astype(jnp.bfloat16).astype(jnp.float32)
    tnl = tn - tnh
    taug = jnp.concatenate(
        [th, th, tl, tnh, tnl, jnp.ones((T, 2), jnp.float32),
         jnp.zeros((T, 4), jnp.float32)], axis=1)
    d2t = jax.lax.dot_general(caug_ref[...], taug, (((1,), (1,)), ((), ())),
                              preferred_element_type=jnp.float32)

    moff = pl.multiple_of((i % per_row) * T, T)
    mlane = m_ref[pl.ds(i // per_row, 1), pl.ds(moff, T)] == 1
    colmin = jnp.min(d2t, axis=0, keepdims=True)
    vlane = jnp.where(mlane, colmin, jnp.inf)
    v = jnp.min(vlane)

    mi = mlane.astype(jnp.int32)
    cnt_here = cnt_ref[0, 0]
    cnt_ref[0, 0] = cnt_here + jnp.sum(mi)

    @pl.when(v < best_ref[0, 0])
    def _update():
        li = jax.lax.broadcasted_iota(jnp.int32, (1, T), 1)
        row = jnp.min(jnp.where(vlane == v, li, jnp.int32(2**31 - 1)))
        colv = jnp.min(jnp.where(li == row, d2t, jnp.inf), axis=1,
                       keepdims=True)
        si = jax.lax.broadcasted_iota(jnp.int32, (NC, 1), 0)
        col = jnp.min(jnp.where(colv == v, si, jnp.int32(2**31 - 1)))
        lrank = jnp.sum(jnp.where(li < row, mi, 0))
        best_ref[0, 0] = v
        lab_ref[0, 0] = (cnt_here + lrank) * NC + col

    @pl.when(i == nt - 1)
    def _fin():
        out_ref[0, 0] = lab_ref[0, 0]


def kernel(input_values, mask_time_indices, W, code_book):
    Bv, Lv, D = input_values.shape
    N = Bv * Lv
    K = W.shape[0]
    NC = code_book.shape[0]
    T = 2048
    while Lv % T:
        T //= 2
    pr = Lv // T
    out = pl.pallas_call(
        _rpq_kernel,
        grid=(N // T,),
        in_specs=[
            pl.BlockSpec((1, T, D), lambda i: (i // pr, i % pr, 0)),
            pl.BlockSpec((Bv, Lv), lambda i: (0, 0)),
            pl.BlockSpec((K, D), lambda i: (0, 0)),
            pl.BlockSpec((NC, K), lambda i: (0, 0)),
        ],
        out_specs=pl.BlockSpec((1, 1), lambda i: (0, 0),
                               memory_space=pltpu.SMEM),
        out_shape=jax.ShapeDtypeStruct((1, 1), jnp.int32),
        scratch_shapes=[
            pltpu.SMEM((1, 1), jnp.float32),
            pltpu.SMEM((1, 1), jnp.int32),
            pltpu.SMEM((1, 1), jnp.int32),
            pltpu.VMEM((NC, 3 * K + 8), jnp.float32),
        ],
    )(input_values, mask_time_indices, W, code_book)
    return out[0, 0]


# speedup vs baseline: 4.0677x; 2.9299x over previous
import jax
import jax.numpy as jnp
from jax.experimental import pallas as pl
from jax.experimental.pallas import tpu as pltpu


def _rpq_kernel(x_ref, m_ref, w_ref, cbt_ref, out_ref,
                best_ref, lab_ref, cnt_ref, caug_ref):
    i = pl.program_id(0)
    nt = pl.num_programs(0)
    T = x_ref.shape[2]
    K, NC = cbt_ref.shape
    Lv = m_ref.shape[1]
    per_row = Lv // T

    @pl.when(i == 0)
    def _init():
        best_ref[0, 0] = jnp.inf
        lab_ref[0, 0] = 0
        cnt_ref[0, 0] = 0
        cbt = cbt_ref[...]
        ch = cbt.astype(jnp.bfloat16).astype(jnp.float32)
        cl = cbt - ch
        cn = jnp.sum(cbt * cbt, axis=0, keepdims=True)
        cnh = cn.astype(jnp.bfloat16).astype(jnp.float32)
        cnl = cn - cnh
        caug_ref[...] = jnp.concatenate(
            [ch * -2.0, cl * -2.0, ch * -2.0,
             jnp.ones((2, NC), jnp.float32), cnh, cnl,
             jnp.zeros((4, NC), jnp.float32)], axis=0)

    xt = x_ref[...].reshape(x_ref.shape[1], T)
    tt = jax.lax.dot_general(w_ref[...], xt, (((1,), (0,)), ((), ())),
                             preferred_element_type=jnp.float32)
    th = tt.astype(jnp.bfloat16).astype(jnp.float32)
    tl = tt - th
    tn = jnp.sum(tt * tt, axis=0, keepdims=True)
    tnh = tn.astype(jnp.bfloat16).astype(jnp.float32)
    tnl = tn - tnh
    taug = jnp.concatenate(
        [th, th, tl, tnh, tnl, jnp.ones((2, T), jnp.float32),
         jnp.zeros((4, T), jnp.float32)], axis=0)
    d2t = jax.lax.dot_general(caug_ref[...], taug, (((0,), (0,)), ((), ())),
                              preferred_element_type=jnp.float32)

    moff = pl.multiple_of((i % per_row) * T, T)
    mlane = m_ref[pl.ds(i // per_row, 1), pl.ds(moff, T)] == 1
    colmin = jnp.min(d2t, axis=0, keepdims=True)
    vlane = jnp.where(mlane, colmin, jnp.inf)
    v = jnp.min(vlane)

    mi = mlane.astype(jnp.int32)
    cnt_here = cnt_ref[0, 0]
    cnt_ref[0, 0] = cnt_here + jnp.sum(mi)

    @pl.when(v < best_ref[0, 0])
    def _update():
        li = jax.lax.broadcasted_iota(jnp.int32, (1, T), 1)
        row = jnp.min(jnp.where(vlane == v, li, jnp.int32(2**31 - 1)))
        colv = jnp.min(jnp.where(li == row, d2t, jnp.inf), axis=1,
                       keepdims=True)
        si = jax.lax.broadcasted_iota(jnp.int32, (NC, 1), 0)
        col = jnp.min(jnp.where(colv == v, si, jnp.int32(2**31 - 1)))
        lrank = jnp.sum(jnp.where(li < row, mi, 0))
        best_ref[0, 0] = v
        lab_ref[0, 0] = (cnt_here + lrank) * NC + col

    @pl.when(i == nt - 1)
    def _fin():
        out_ref[0, 0] = lab_ref[0, 0]


def kernel(input_values, mask_time_indices, W, code_book):
    Bv, Lv, D = input_values.shape
    N = Bv * Lv
    K = W.shape[0]
    NC = code_book.shape[0]
    T = 2048
    while Lv % T:
        T //= 2
    pr = Lv // T
    xt = jnp.swapaxes(input_values, 1, 2)
    cbt = code_book.T
    out = pl.pallas_call(
        _rpq_kernel,
        grid=(N // T,),
        in_specs=[
            pl.BlockSpec((1, D, T), lambda i: (i // pr, 0, i % pr)),
            pl.BlockSpec((Bv, Lv), lambda i: (0, 0)),
            pl.BlockSpec((K, D), lambda i: (0, 0)),
            pl.BlockSpec((K, NC), lambda i: (0, 0)),
        ],
        out_specs=pl.BlockSpec((1, 1), lambda i: (0, 0),
                               memory_space=pltpu.SMEM),
        out_shape=jax.ShapeDtypeStruct((1, 1), jnp.int32),
        scratch_shapes=[
            pltpu.SMEM((1, 1), jnp.float32),
            pltpu.SMEM((1, 1), jnp.int32),
            pltpu.SMEM((1, 1), jnp.int32),
            pltpu.VMEM((3 * K + 8, NC), jnp.float32),
        ],
    )(xt, mask_time_indices, W, cbt)
    return out[0, 0]
